# baseline (speedup 1.0000x reference)
import jax
import jax.numpy as jnp
from jax import lax
from jax.experimental import pallas as pl
from jax.experimental.pallas import tpu as pltpu
from jax.experimental.pallas import tpu_sc as plsc

B = 16384
D = 16
NC, NS, L = 2, 16, 16
NW = NC * NS
B_PER_W = B // NW
CHUNK = 32
N_CHUNKS = B_PER_W // CHUNK
GROUPS_PER_CHUNK = CHUNK // L

GUMBEL_BETA = 0.01
INV_GB = 100.0
EG2 = 2.0 * 0.5772156649015329 * GUMBEL_BETA
TINY = 1.1754943508222875e-38
LN2 = 0.6931471805599453
SQRT_HALF_BITS = 0x3F3504F3

_LOG_C = (-1.4097389054723575e-11, 0.9999998807907104, -0.49999991059303284,
          0.3333507776260376, -0.2500225603580475, 0.19936639070510864,
          -0.16551056504249573, 0.15102536976337433, -0.14478063583374023,
          0.08491219580173492)


def _log_poly(u):
    acc = jnp.full_like(u, _LOG_C[-1])
    for c in reversed(_LOG_C[:-1]):
        acc = acc * u + c
    return acc


def _fast_log(x):
    bits = plsc.bitcast(x, jnp.int32)
    k = lax.shift_right_arithmetic(bits - SQRT_HALF_BITS, 23)
    m = plsc.bitcast(bits - lax.shift_left(k, 23), jnp.float32)
    return k.astype(jnp.float32) * LN2 + _log_poly(m - 1.0)


def _log1p_exp(a):
    z = jnp.exp(a)
    s = 1.0 + z
    r = z - (s - 1.0)
    return _fast_log(s) + r


def _log_softplus(x):
    sp = jnp.maximum(x, 0.0) + _log1p_exp(-jnp.abs(x))
    return _fast_log(jnp.maximum(sp, TINY))


V = 1000000
R = 100000
EBV = 1536
EBR = 768
V_BATCHES = 651
R_BATCHES = 130
R_EXTRA0 = R_BATCHES * EBR
R_OUT_ROWS = R // 8 + 4


def _conv_batch(in_v, out_v, lane, n_rows):
    assert n_rows % 2 == 0
    lane16 = (lane & 7) * 16
    lane8 = lax.shift_right_logical(lane, 3)

    @plsc.parallel_loop(0, n_rows // 2, unroll=4)
    def blocks(bi):
        rows = bi * 2 + lane8
        ecol = bi * 16 + lane
        for k in range(16):
            rotk = (lane + k) & 15
            v = plsc.load_gather(in_v, [rotk, ecol])
            plsc.store_scatter(out_v, [rows, lane16 + rotk], v)


def _relayout_body(tm, td, trh, tsh, trt, tst,
                   xm, xd, xrh, xsh, xrt, xst,
                   om, od, orh, osh, ort, ost,
                   in0, in1, out0, out1, is0, is1, os0, os1):
    w = lax.axis_index("s") * NC + lax.axis_index("c")
    lane = lax.iota(jnp.int32, L)
    ins, outs, isems, osems = (in0, in1), (out0, out1), (is0, is1), (os0, os1)

    def conv_table(T, O, start, cnt, eb):
        end = start + cnt
        nr = eb // 8

        for p in range(2):
            @pl.when(cnt > p)
            def _():
                pltpu.async_copy(T.at[:, pl.ds((start + p) * eb, eb)],
                                 ins[p].at[:, pl.ds(0, eb)], isems[p])

        def pair(i, _):
            for p in range(2):
                b = start + 2 * i + p

                @pl.when(b < end)
                def _():
                    pltpu.make_async_copy(T.at[:, pl.ds(0, eb)],
                                          ins[p].at[:, pl.ds(0, eb)],
                                          isems[p]).wait()

                    @pl.when(i > 0)
                    def _():
                        pltpu.make_async_copy(
                            outs[p].at[pl.ds(0, nr), :],
                            O.at[pl.ds(0, nr), :], osems[p]).wait()

                    _conv_batch(ins[p], outs[p], lane, nr)
                    pltpu.async_copy(outs[p].at[pl.ds(0, nr), :],
                                     O.at[pl.ds(b * nr, nr), :], osems[p])

                    @pl.when(b + 2 < end)
                    def _():
                        pltpu.async_copy(T.at[:, pl.ds((b + 2) * eb, eb)],
                                         ins[p].at[:, pl.ds(0, eb)], isems[p])
            return ()

        lax.fori_loop(0, (cnt + 1) // 2, pair, ())
        for p in range(2):
            @pl.when(cnt > p)
            def _():
                pltpu.make_async_copy(outs[p].at[pl.ds(0, nr), :],
                                      O.at[pl.ds(0, nr), :], osems[p]).wait()

    vs = w * 20 + jnp.minimum(w, 11)
    vc = 20 + (w < 11).astype(jnp.int32)
    rs = w * 4 + jnp.minimum(w, 2)
    rc = 4 + (w < 2).astype(jnp.int32)
    conv_table(tm, om, vs, vc, EBV)
    conv_table(td, od, vs, vc, EBV)
    for T, O in ((trh, orh), (tsh, osh), (trt, ort), (tst, ost)):
        conv_table(T, O, rs, rc, EBR)

    @pl.when(w == NW - 1)
    def _():
        for T, O in ((trh, orh), (tsh, osh), (trt, ort), (tst, ost)):
            pltpu.async_copy(T.at[:, pl.ds(R_EXTRA0, 128)],
                             in0.at[:, pl.ds(0, 128)], is0)
            pltpu.make_async_copy(T.at[:, pl.ds(0, 128)],
                                  in0.at[:, pl.ds(0, 128)], is0).wait()
            _conv_batch(in0, out0, lane, 16)
            pltpu.async_copy(out0.at[pl.ds(0, 16), :],
                             O.at[pl.ds(R_EXTRA0 // 8, 16), :], os0)
            pltpu.make_async_copy(out0.at[pl.ds(0, 16), :],
                                  O.at[pl.ds(0, 16), :], os0).wait()
        for X, O, t0 in ((xm, om, V // 8 - 8), (xd, od, V // 8 - 8),
                         (xrh, orh, R_OUT_ROWS - 8), (xsh, osh, R_OUT_ROWS - 8),
                         (xrt, ort, R_OUT_ROWS - 8), (xst, ost, R_OUT_ROWS - 8)):
            pltpu.async_copy(X, in0.at[:, pl.ds(0, 128)], is0)
            pltpu.make_async_copy(X, in0.at[:, pl.ds(0, 128)], is0).wait()
            _conv_batch(in0, out0, lane, 8)
            pltpu.async_copy(out0.at[pl.ds(0, 8), :],
                             O.at[pl.ds(t0, 8), :], os0)
            pltpu.make_async_copy(out0.at[pl.ds(0, 8), :],
                                  O.at[pl.ds(0, 8), :], os0).wait()


@jax.jit
def _relayout_tables(tm, td, trh, tsh, trt, tst):
    tails = [jnp.pad(t[:, -n:], ((0, 0), (0, 128 - n)))
             for t, n in ((tm, 64), (td, 64), (trh, 32), (tsh, 32),
                          (trt, 32), (tst, 32))]
    mesh = plsc.VectorSubcoreMesh(core_axis_name="c", subcore_axis_name="s",
                                  num_cores=NC, num_subcores=NS)
    run = pl.kernel(
        _relayout_body,
        out_type=[jax.ShapeDtypeStruct((V // 8, 128), jnp.float32)] * 2
        + [jax.ShapeDtypeStruct((R_OUT_ROWS, 128), jnp.float32)] * 4,
        mesh=mesh,
        compiler_params=pltpu.CompilerParams(needs_layout_passes=False),
        scratch_types=[
            pltpu.VMEM((16, EBV), jnp.float32),
            pltpu.VMEM((16, EBV), jnp.float32),
            pltpu.VMEM((EBV // 8, 128), jnp.float32),
            pltpu.VMEM((EBV // 8, 128), jnp.float32),
            pltpu.SemaphoreType.DMA, pltpu.SemaphoreType.DMA,
            pltpu.SemaphoreType.DMA, pltpu.SemaphoreType.DMA,
        ],
    )
    return run(tm, td, trh, tsh, trt, tst, *tails)


def _sc_body(h_hbm, r_hbm, t_hbm, mine_hbm, dele_hbm, rth_hbm, rsh_hbm,
             rtt_hbm, rst_hbm, out_hbm,
             hids, rids, tids, hi_h, hi_r, hi_t,
             b_mnh, b_dlh, b_mnt, b_dlt, b_trh, b_sch, b_trt, b_sct,
             out_v, sem):
    wid = lax.axis_index("s") * NC + lax.axis_index("c")
    base = wid * B_PER_W

    pltpu.sync_copy(h_hbm.at[pl.ds(base, B_PER_W)], hids)
    pltpu.sync_copy(r_hbm.at[pl.ds(base, B_PER_W)], rids)
    pltpu.sync_copy(t_hbm.at[pl.ds(base, B_PER_W)], tids)

    for c in range(B_PER_W // L):
        sl = pl.ds((c % GROUPS_PER_CHUNK) * L, L)
        hi_h[c // GROUPS_PER_CHUNK, sl] = lax.shift_right_logical(
            hids[pl.ds(c * L, L)], 3)
        hi_r[c // GROUPS_PER_CHUNK, sl] = lax.shift_right_logical(
            rids[pl.ds(c * L, L)], 3)
        hi_t[c // GROUPS_PER_CHUNK, sl] = lax.shift_right_logical(
            tids[pl.ds(c * L, L)], 3)

    lane = lax.iota(jnp.int32, L)

    def chunk(k, _):
        copies = [
            pltpu.async_copy(mine_hbm.at[hi_h.at[k]], b_mnh, sem),
            pltpu.async_copy(dele_hbm.at[hi_h.at[k]], b_dlh, sem),
            pltpu.async_copy(mine_hbm.at[hi_t.at[k]], b_mnt, sem),
            pltpu.async_copy(dele_hbm.at[hi_t.at[k]], b_dlt, sem),
            pltpu.async_copy(rth_hbm.at[hi_r.at[k]], b_trh, sem),
            pltpu.async_copy(rsh_hbm.at[hi_r.at[k]], b_sch, sem),
            pltpu.async_copy(rtt_hbm.at[hi_r.at[k]], b_trt, sem),
            pltpu.async_copy(rst_hbm.at[hi_r.at[k]], b_sct, sem),
        ]
        for cp in copies:
            cp.wait()

        def group(gg, _):
            off = k * CHUNK + gg * L
            rows = gg * L + lane
            col_h = (hids[pl.ds(off, L)] & 7) * 16
            col_r = (rids[pl.ds(off, L)] & 7) * 16
            col_t = (tids[pl.ds(off, L)] & 7) * 16

            def dim(d, acc):
                rot = (lane + d) & 15
                ch = col_h + rot
                cr = col_r + rot
                ct = col_t + rot
                mh = plsc.load_gather(b_mnh, [rows, ch])
                dh = plsc.load_gather(b_dlh, [rows, ch])
                mt = plsc.load_gather(b_mnt, [rows, ct])
                dt = plsc.load_gather(b_dlt, [rows, ct])
                th = plsc.load_gather(b_trh, [rows, cr])
                sh = plsc.load_gather(b_sch, [rows, cr])
                tt = plsc.load_gather(b_trt, [rows, cr])
                st = plsc.load_gather(b_sct, [rows, cr])

                h_mn = mh + th
                h_mx = h_mn + jnp.exp(dh) * jnp.maximum(sh, 0.0)
                t_mn = mt + tt
                t_mx = t_mn + jnp.exp(dt) * jnp.maximum(st, 0.0)

                i_mn = jnp.maximum(h_mn, t_mn) + GUMBEL_BETA * _log1p_exp(
                    -jnp.abs(h_mn - t_mn) * INV_GB)
                i_mx = jnp.minimum(h_mx, t_mx) - GUMBEL_BETA * _log1p_exp(
                    -jnp.abs(h_mx - t_mx) * INV_GB)

                acc += _log_softplus((i_mx - i_mn) - EG2)
                acc -= _log_softplus((t_mx - t_mn) - EG2)
                return acc

            out_v[pl.ds(off, L)] = plsc.parallel_loop(
                0, D, unroll=2, carry=jnp.zeros((L,), jnp.float32))(dim)
            return ()

        lax.fori_loop(0, GROUPS_PER_CHUNK, group, ())
        return ()

    lax.fori_loop(0, N_CHUNKS, chunk, ())
    pltpu.sync_copy(out_v, out_hbm.at[pl.ds(base, B_PER_W)])


@jax.jit
def _bi_gumbel_box_sc(h_ids, r_ids, t_ids, mine_blk, dele_blk,
                      rth_blk, rsh_blk, rtt_blk, rst_blk):
    mesh = plsc.VectorSubcoreMesh(core_axis_name="c", subcore_axis_name="s",
                                  num_cores=NC, num_subcores=NS)
    run = pl.kernel(
        _sc_body,
        out_type=jax.ShapeDtypeStruct((B,), jnp.float32),
        mesh=mesh,
        compiler_params=pltpu.CompilerParams(needs_layout_passes=False),
        scratch_types=[
            pltpu.VMEM((B_PER_W,), jnp.int32),
            pltpu.VMEM((B_PER_W,), jnp.int32),
            pltpu.VMEM((B_PER_W,), jnp.int32),
            pltpu.VMEM((N_CHUNKS, CHUNK), jnp.int32),
            pltpu.VMEM((N_CHUNKS, CHUNK), jnp.int32),
            pltpu.VMEM((N_CHUNKS, CHUNK), jnp.int32),
        ] + [pltpu.VMEM((CHUNK, 128), jnp.float32)] * 8 + [
            pltpu.VMEM((B_PER_W,), jnp.float32),
            pltpu.SemaphoreType.DMA,
        ],
    )
    return run(h_ids, r_ids, t_ids, mine_blk, dele_blk,
               rth_blk, rsh_blk, rtt_blk, rst_blk)


def kernel(ids, probs, min_embedding, delta_embedding, rel_trans_for_head,
           rel_scale_for_head, rel_trans_for_tail, rel_scale_for_tail):
    h_ids = ids[:, 0].astype(jnp.int32)
    r_ids = ids[:, 1].astype(jnp.int32)
    t_ids = ids[:, 2].astype(jnp.int32)
    tables = _relayout_tables(
        min_embedding.T, delta_embedding.T, rel_trans_for_head.T,
        rel_scale_for_head.T, rel_trans_for_tail.T, rel_scale_for_tail.T)
    log_prob = _bi_gumbel_box_sc(h_ids, r_ids, t_ids, *tables)
    return (log_prob, probs)


# speedup vs baseline: 1.0279x; 1.0279x over previous
import jax
import jax.numpy as jnp
from jax import lax
from jax.experimental import pallas as pl
from jax.experimental.pallas import tpu as pltpu
from jax.experimental.pallas import tpu_sc as plsc

B = 16384
D = 16
NC, NS, L = 2, 16, 16
NW = NC * NS
B_PER_W = B // NW
CHUNK = 32
N_CHUNKS = B_PER_W // CHUNK
GROUPS_PER_CHUNK = CHUNK // L

GUMBEL_BETA = 0.01
INV_GB = 100.0
EG2 = 2.0 * 0.5772156649015329 * GUMBEL_BETA
TINY = 1.1754943508222875e-38
LN2 = 0.6931471805599453
SQRT_HALF_BITS = 0x3F3504F3

_LOG_C = (-1.4097389054723575e-11, 0.9999998807907104, -0.49999991059303284,
          0.3333507776260376, -0.2500225603580475, 0.19936639070510864,
          -0.16551056504249573, 0.15102536976337433, -0.14478063583374023,
          0.08491219580173492)


def _log_poly(u):
    acc = jnp.full_like(u, _LOG_C[-1])
    for c in reversed(_LOG_C[:-1]):
        acc = acc * u + c
    return acc


def _fast_log(x):
    bits = plsc.bitcast(x, jnp.int32)
    k = lax.shift_right_arithmetic(bits - SQRT_HALF_BITS, 23)
    m = plsc.bitcast(bits - lax.shift_left(k, 23), jnp.float32)
    return k.astype(jnp.float32) * LN2 + _log_poly(m - 1.0)


def _log1p_exp(a):
    z = jnp.exp(a)
    s = 1.0 + z
    r = z - (s - 1.0)
    return _fast_log(s) + r


def _log_softplus(x):
    sp = jnp.maximum(x, 0.0) + _log1p_exp(-jnp.abs(x))
    return _fast_log(jnp.maximum(sp, TINY))


V = 1000000
R = 100000
EBV = 1536
EBR = 768
V_BATCHES = 651
R_BATCHES = 130
R_EXTRA0 = R_BATCHES * EBR
R_OUT_ROWS = R // 8 + 4


def _conv_batch(in_v, out_v, lane, n_rows):
    assert n_rows % 2 == 0
    lane16 = (lane & 7) * 16
    lane8 = lax.shift_right_logical(lane, 3)

    @plsc.parallel_loop(0, n_rows // 2, unroll=2)
    def blocks(bi):
        rows = bi * 2 + lane8
        ecol = bi * 16 + lane
        for k in range(16):
            rotk = (lane + k) & 15
            v = plsc.load_gather(in_v, [rotk, ecol])
            plsc.store_scatter(out_v, [rows, lane16 + rotk], v)


def _relayout_body(tm, td, trh, tsh, trt, tst,
                   xm, xd, xrh, xsh, xrt, xst,
                   om, od, orh, osh, ort, ost,
                   in0, in1, out0, out1, is0, is1, os0, os1):
    w = lax.axis_index("s") * NC + lax.axis_index("c")
    lane = lax.iota(jnp.int32, L)
    ins, outs, isems, osems = (in0, in1), (out0, out1), (is0, is1), (os0, os1)

    def conv_table(T, O, start, cnt, eb):
        end = start + cnt
        nr = eb // 8

        for p in range(2):
            @pl.when(cnt > p)
            def _():
                pltpu.async_copy(T.at[:, pl.ds((start + p) * eb, eb)],
                                 ins[p].at[:, pl.ds(0, eb)], isems[p])

        def pair(i, _):
            for p in range(2):
                b = start + 2 * i + p

                @pl.when(b < end)
                def _():
                    pltpu.make_async_copy(T.at[:, pl.ds(0, eb)],
                                          ins[p].at[:, pl.ds(0, eb)],
                                          isems[p]).wait()

                    @pl.when(i > 0)
                    def _():
                        pltpu.make_async_copy(
                            outs[p].at[pl.ds(0, nr), :],
                            O.at[pl.ds(0, nr), :], osems[p]).wait()

                    _conv_batch(ins[p], outs[p], lane, nr)
                    pltpu.async_copy(outs[p].at[pl.ds(0, nr), :],
                                     O.at[pl.ds(b * nr, nr), :], osems[p])

                    @pl.when(b + 2 < end)
                    def _():
                        pltpu.async_copy(T.at[:, pl.ds((b + 2) * eb, eb)],
                                         ins[p].at[:, pl.ds(0, eb)], isems[p])
            return ()

        lax.fori_loop(0, (cnt + 1) // 2, pair, ())
        for p in range(2):
            @pl.when(cnt > p)
            def _():
                pltpu.make_async_copy(outs[p].at[pl.ds(0, nr), :],
                                      O.at[pl.ds(0, nr), :], osems[p]).wait()

    vs = w * 20 + jnp.minimum(w, 11)
    vc = 20 + (w < 11).astype(jnp.int32)
    rs = w * 4 + jnp.minimum(w, 2)
    rc = 4 + (w < 2).astype(jnp.int32)
    conv_table(tm, om, vs, vc, EBV)
    conv_table(td, od, vs, vc, EBV)
    for T, O in ((trh, orh), (tsh, osh), (trt, ort), (tst, ost)):
        conv_table(T, O, rs, rc, EBR)

    @pl.when(w == NW - 1)
    def _():
        for T, O in ((trh, orh), (tsh, osh), (trt, ort), (tst, ost)):
            pltpu.async_copy(T.at[:, pl.ds(R_EXTRA0, 128)],
                             in0.at[:, pl.ds(0, 128)], is0)
            pltpu.make_async_copy(T.at[:, pl.ds(0, 128)],
                                  in0.at[:, pl.ds(0, 128)], is0).wait()
            _conv_batch(in0, out0, lane, 16)
            pltpu.async_copy(out0.at[pl.ds(0, 16), :],
                             O.at[pl.ds(R_EXTRA0 // 8, 16), :], os0)
            pltpu.make_async_copy(out0.at[pl.ds(0, 16), :],
                                  O.at[pl.ds(0, 16), :], os0).wait()
        for X, O, t0 in ((xm, om, V // 8 - 8), (xd, od, V // 8 - 8),
                         (xrh, orh, R_OUT_ROWS - 8), (xsh, osh, R_OUT_ROWS - 8),
                         (xrt, ort, R_OUT_ROWS - 8), (xst, ost, R_OUT_ROWS - 8)):
            pltpu.async_copy(X, in0.at[:, pl.ds(0, 128)], is0)
            pltpu.make_async_copy(X, in0.at[:, pl.ds(0, 128)], is0).wait()
            _conv_batch(in0, out0, lane, 8)
            pltpu.async_copy(out0.at[pl.ds(0, 8), :],
                             O.at[pl.ds(t0, 8), :], os0)
            pltpu.make_async_copy(out0.at[pl.ds(0, 8), :],
                                  O.at[pl.ds(0, 8), :], os0).wait()


@jax.jit
def _relayout_tables(tm, td, trh, tsh, trt, tst):
    tails = [jnp.pad(t[:, -n:], ((0, 0), (0, 128 - n)))
             for t, n in ((tm, 64), (td, 64), (trh, 32), (tsh, 32),
                          (trt, 32), (tst, 32))]
    mesh = plsc.VectorSubcoreMesh(core_axis_name="c", subcore_axis_name="s",
                                  num_cores=NC, num_subcores=NS)
    run = pl.kernel(
        _relayout_body,
        out_type=[jax.ShapeDtypeStruct((V // 8, 128), jnp.float32)] * 2
        + [jax.ShapeDtypeStruct((R_OUT_ROWS, 128), jnp.float32)] * 4,
        mesh=mesh,
        compiler_params=pltpu.CompilerParams(needs_layout_passes=False),
        scratch_types=[
            pltpu.VMEM((16, EBV), jnp.float32),
            pltpu.VMEM((16, EBV), jnp.float32),
            pltpu.VMEM((EBV // 8, 128), jnp.float32),
            pltpu.VMEM((EBV // 8, 128), jnp.float32),
            pltpu.SemaphoreType.DMA, pltpu.SemaphoreType.DMA,
            pltpu.SemaphoreType.DMA, pltpu.SemaphoreType.DMA,
        ],
    )
    return run(tm, td, trh, tsh, trt, tst, *tails)


def _sc_body(h_hbm, r_hbm, t_hbm, mine_hbm, dele_hbm, rth_hbm, rsh_hbm,
             rtt_hbm, rst_hbm, out_hbm,
             hids, rids, tids, hi_h, hi_r, hi_t,
             b_mnh, b_dlh, b_mnt, b_dlt, b_trh, b_sch, b_trt, b_sct,
             out_v, sem):
    wid = lax.axis_index("s") * NC + lax.axis_index("c")
    base = wid * B_PER_W

    pltpu.sync_copy(h_hbm.at[pl.ds(base, B_PER_W)], hids)
    pltpu.sync_copy(r_hbm.at[pl.ds(base, B_PER_W)], rids)
    pltpu.sync_copy(t_hbm.at[pl.ds(base, B_PER_W)], tids)

    for c in range(B_PER_W // L):
        sl = pl.ds((c % GROUPS_PER_CHUNK) * L, L)
        hi_h[c // GROUPS_PER_CHUNK, sl] = lax.shift_right_logical(
            hids[pl.ds(c * L, L)], 3)
        hi_r[c // GROUPS_PER_CHUNK, sl] = lax.shift_right_logical(
            rids[pl.ds(c * L, L)], 3)
        hi_t[c // GROUPS_PER_CHUNK, sl] = lax.shift_right_logical(
            tids[pl.ds(c * L, L)], 3)

    lane = lax.iota(jnp.int32, L)

    def chunk(k, _):
        copies = [
            pltpu.async_copy(mine_hbm.at[hi_h.at[k]], b_mnh, sem),
            pltpu.async_copy(dele_hbm.at[hi_h.at[k]], b_dlh, sem),
            pltpu.async_copy(mine_hbm.at[hi_t.at[k]], b_mnt, sem),
            pltpu.async_copy(dele_hbm.at[hi_t.at[k]], b_dlt, sem),
            pltpu.async_copy(rth_hbm.at[hi_r.at[k]], b_trh, sem),
            pltpu.async_copy(rsh_hbm.at[hi_r.at[k]], b_sch, sem),
            pltpu.async_copy(rtt_hbm.at[hi_r.at[k]], b_trt, sem),
            pltpu.async_copy(rst_hbm.at[hi_r.at[k]], b_sct, sem),
        ]
        for cp in copies:
            cp.wait()

        def group(gg, _):
            off = k * CHUNK + gg * L
            rows = gg * L + lane
            col_h = (hids[pl.ds(off, L)] & 7) * 16
            col_r = (rids[pl.ds(off, L)] & 7) * 16
            col_t = (tids[pl.ds(off, L)] & 7) * 16

            def dim(d, acc):
                rot = (lane + d) & 15
                ch = col_h + rot
                cr = col_r + rot
                ct = col_t + rot
                mh = plsc.load_gather(b_mnh, [rows, ch])
                dh = plsc.load_gather(b_dlh, [rows, ch])
                mt = plsc.load_gather(b_mnt, [rows, ct])
                dt = plsc.load_gather(b_dlt, [rows, ct])
                th = plsc.load_gather(b_trh, [rows, cr])
                sh = plsc.load_gather(b_sch, [rows, cr])
                tt = plsc.load_gather(b_trt, [rows, cr])
                st = plsc.load_gather(b_sct, [rows, cr])

                h_mn = mh + th
                h_mx = h_mn + jnp.exp(dh) * jnp.maximum(sh, 0.0)
                t_mn = mt + tt
                t_mx = t_mn + jnp.exp(dt) * jnp.maximum(st, 0.0)

                i_mn = jnp.maximum(h_mn, t_mn) + GUMBEL_BETA * _log1p_exp(
                    -jnp.abs(h_mn - t_mn) * INV_GB)
                i_mx = jnp.minimum(h_mx, t_mx) - GUMBEL_BETA * _log1p_exp(
                    -jnp.abs(h_mx - t_mx) * INV_GB)

                acc += _log_softplus((i_mx - i_mn) - EG2)
                acc -= _log_softplus((t_mx - t_mn) - EG2)
                return acc

            out_v[pl.ds(off, L)] = lax.fori_loop(
                0, D, dim, jnp.zeros((L,), jnp.float32))
            return ()

        lax.fori_loop(0, GROUPS_PER_CHUNK, group, ())
        return ()

    lax.fori_loop(0, N_CHUNKS, chunk, ())
    pltpu.sync_copy(out_v, out_hbm.at[pl.ds(base, B_PER_W)])


@jax.jit
def _bi_gumbel_box_sc(h_ids, r_ids, t_ids, mine_blk, dele_blk,
                      rth_blk, rsh_blk, rtt_blk, rst_blk):
    mesh = plsc.VectorSubcoreMesh(core_axis_name="c", subcore_axis_name="s",
                                  num_cores=NC, num_subcores=NS)
    run = pl.kernel(
        _sc_body,
        out_type=jax.ShapeDtypeStruct((B,), jnp.float32),
        mesh=mesh,
        compiler_params=pltpu.CompilerParams(needs_layout_passes=False),
        scratch_types=[
            pltpu.VMEM((B_PER_W,), jnp.int32),
            pltpu.VMEM((B_PER_W,), jnp.int32),
            pltpu.VMEM((B_PER_W,), jnp.int32),
            pltpu.VMEM((N_CHUNKS, CHUNK), jnp.int32),
            pltpu.VMEM((N_CHUNKS, CHUNK), jnp.int32),
            pltpu.VMEM((N_CHUNKS, CHUNK), jnp.int32),
        ] + [pltpu.VMEM((CHUNK, 128), jnp.float32)] * 8 + [
            pltpu.VMEM((B_PER_W,), jnp.float32),
            pltpu.SemaphoreType.DMA,
        ],
    )
    return run(h_ids, r_ids, t_ids, mine_blk, dele_blk,
               rth_blk, rsh_blk, rtt_blk, rst_blk)


def kernel(ids, probs, min_embedding, delta_embedding, rel_trans_for_head,
           rel_scale_for_head, rel_trans_for_tail, rel_scale_for_tail):
    h_ids = ids[:, 0].astype(jnp.int32)
    r_ids = ids[:, 1].astype(jnp.int32)
    t_ids = ids[:, 2].astype(jnp.int32)
    tables = _relayout_tables(
        min_embedding.T, delta_embedding.T, rel_trans_for_head.T,
        rel_scale_for_head.T, rel_trans_for_tail.T, rel_scale_for_tail.T)
    log_prob = _bi_gumbel_box_sc(h_ids, r_ids, t_ids, *tables)
    return (log_prob, probs)
